# baseline (speedup 1.0000x reference)
import functools

import jax
import jax.numpy as jnp
from jax import lax
from jax.experimental import pallas as pl
from jax.experimental.pallas import tpu as pltpu
from jax.experimental.pallas import tpu_sc as plsc

N = 10000
E = 320000
F = 128
R = 8
NBASES = 4

NC = 2
NS = 16
NTILES = NC * NS
CH = 80
NCH = 128
EPT = NCH * CH
EPAD = NTILES * EPT
NACC = N + 8
RPT = 624
ZTAIL = NACC - NS * RPT
OTAIL = N - NS * RPT

BLK = 2000
NBLK = N // BLK


def _rel9_body(w_ref, wc_ref, lw_ref, out_ref):
    for r in range(R):
        acc = wc_ref[r, 0] * w_ref[0]
        for b in range(1, NBASES):
            acc = acc + wc_ref[r, b] * w_ref[b]
        out_ref[r] = acc
    out_ref[R] = lw_ref[...]


def _rel9(weight, w_comp, loop_weight):
    return pl.pallas_call(
        _rel9_body,
        out_shape=jax.ShapeDtypeStruct((R + 1, F, F), jnp.float32),
        in_specs=[
            pl.BlockSpec(memory_space=pltpu.MemorySpace.VMEM),
            pl.BlockSpec(memory_space=pltpu.MemorySpace.SMEM),
            pl.BlockSpec(memory_space=pltpu.MemorySpace.VMEM),
        ],
        out_specs=pl.BlockSpec(memory_space=pltpu.MemorySpace.VMEM),
    )(weight, w_comp, loop_weight)


def _hall_body(x_ref, rel_ref, out_ref):
    out_ref[0] = jnp.dot(x_ref[...], rel_ref[0],
                         preferred_element_type=jnp.float32)


def _hall(x, rel9):
    return pl.pallas_call(
        _hall_body,
        grid=(NBLK, R + 1),
        in_specs=[
            pl.BlockSpec((BLK, F), lambda nb, r: (nb, 0)),
            pl.BlockSpec((1, F, F), lambda nb, r: (r, 0, 0)),
        ],
        out_specs=pl.BlockSpec((1, BLK, F), lambda nb, r: (r, nb, 0)),
        out_shape=jax.ShapeDtypeStruct((R + 1, N, F), jnp.float32),
    )(x, rel9)


def _sc_agg_body(hall_hbm, src_hbm, et_hbm, dst_hbm, zeros_hbm, out_hbm,
                 gidx, didx, buf, acc, gsem):
    c = lax.axis_index("c")
    s = lax.axis_index("s")
    tile = c * NS + s

    pltpu.sync_copy(zeros_hbm.at[pl.ds(s * RPT, RPT)],
                    acc.at[pl.ds(s * RPT, RPT)])

    @pl.when(s == NS - 1)
    def _zero_tail():
        pltpu.sync_copy(zeros_hbm.at[pl.ds(NS * RPT, ZTAIL)],
                        acc.at[pl.ds(NS * RPT, ZTAIL)])

    pltpu.sync_copy(et_hbm.at[tile], gidx)
    pltpu.sync_copy(src_hbm.at[tile], didx)

    def idx_row(j, _):
        def idx_vec(k, _):
            sl = pl.ds(k * 16, 16)
            gidx[j, sl] = gidx[j, sl] * N + didx[j, sl]
            return 0
        return lax.fori_loop(0, CH // 16, idx_vec, 0)

    lax.fori_loop(0, NCH, idx_row, 0)
    pltpu.sync_copy(dst_hbm.at[tile], didx)

    plsc.subcore_barrier()

    def chunk(j, _):
        pltpu.async_copy(hall_hbm.at[gidx.at[j]], buf, gsem).wait()
        pltpu.sync_copy(buf, acc.at[didx.at[j]], add=True)
        return 0

    lax.fori_loop(0, NCH, chunk, 0)

    plsc.subcore_barrier()
    pltpu.sync_copy(acc.at[pl.ds(s * RPT, RPT)],
                    out_hbm.at[c, pl.ds(s * RPT, RPT)])

    @pl.when(s == NS - 1)
    def _write_tail():
        pltpu.sync_copy(acc.at[pl.ds(NS * RPT, OTAIL)],
                        out_hbm.at[c, pl.ds(NS * RPT, OTAIL)])


@functools.cache
def _sc_agg():
    return pl.kernel(
        _sc_agg_body,
        out_type=jax.ShapeDtypeStruct((NC, N, F), jnp.float32),
        mesh=plsc.VectorSubcoreMesh(core_axis_name="c", subcore_axis_name="s",
                                    num_cores=NC, num_subcores=NS),
        scratch_types=[
            pltpu.VMEM((NCH, CH), jnp.int32),
            pltpu.VMEM((NCH, CH), jnp.int32),
            pltpu.VMEM((CH, F), jnp.float32),
            pltpu.VMEM_SHARED((NACC, F), jnp.float32),
            pltpu.SemaphoreType.DMA,
        ],
    )


def _mlp_body(p_ref, self_ref, bias_ref, w1_ref, b1_ref, w2_ref, b2_ref,
              out_ref):
    h = p_ref[0] + p_ref[1] + self_ref[0] + bias_ref[...]
    h = jnp.maximum(
        jnp.dot(h, w1_ref[...], preferred_element_type=jnp.float32)
        + b1_ref[...], 0.0)
    out_ref[...] = jnp.maximum(
        jnp.dot(h, w2_ref[...], preferred_element_type=jnp.float32)
        + b2_ref[...], 0.0)


def _mlp(parts, hall, h_bias, W1, b1, W2, b2):
    return pl.pallas_call(
        _mlp_body,
        grid=(NBLK,),
        in_specs=[
            pl.BlockSpec((NC, BLK, F), lambda nb: (0, nb, 0)),
            pl.BlockSpec((1, BLK, F), lambda nb: (R, nb, 0)),
            pl.BlockSpec((1, F), lambda nb: (0, 0)),
            pl.BlockSpec((F, F), lambda nb: (0, 0)),
            pl.BlockSpec((1, F), lambda nb: (0, 0)),
            pl.BlockSpec((F, F), lambda nb: (0, 0)),
            pl.BlockSpec((1, F), lambda nb: (0, 0)),
        ],
        out_specs=pl.BlockSpec((BLK, F), lambda nb: (nb, 0)),
        out_shape=jax.ShapeDtypeStruct((N, F), jnp.float32),
    )(parts, hall, h_bias.reshape(1, F), W1, b1.reshape(1, F), W2,
      b2.reshape(1, F))


def kernel(x, edge_index, etypes, weight, w_comp, loop_weight, h_bias,
           W1, b1, W2, b2):
    src = edge_index[0].astype(jnp.int32)
    dst = edge_index[1].astype(jnp.int32)
    et = etypes.astype(jnp.int32)

    pad = EPAD - E
    src3 = jnp.concatenate([src, jnp.zeros((pad,), jnp.int32)]) \
        .reshape(NTILES, NCH, CH)
    et3 = jnp.concatenate([et, jnp.zeros((pad,), jnp.int32)]) \
        .reshape(NTILES, NCH, CH)
    pad_dst = N + (jnp.arange(pad, dtype=jnp.int32) % 8)
    dst3 = jnp.concatenate([dst, pad_dst]).reshape(NTILES, NCH, CH)

    rel9 = _rel9(weight, w_comp, loop_weight)
    hall = _hall(x, rel9)
    zeros = jnp.zeros((NACC, F), jnp.float32)
    parts = _sc_agg()(hall.reshape((R + 1) * N, F), src3, et3, dst3, zeros)
    return _mlp(parts, hall, h_bias, W1, b1, W2, b2)


# speedup vs baseline: 3.0570x; 3.0570x over previous
import functools

import jax
import jax.numpy as jnp
from jax import lax
from jax.experimental import pallas as pl
from jax.experimental.pallas import tpu as pltpu
from jax.experimental.pallas import tpu_sc as plsc

N = 10000
E = 320000
F = 128
R = 8
NBASES = 4

NC = 2
NS = 16
NTILES = NC * NS
CH = 80
NCH = 125
EPT = NCH * CH
DBITS = 14
DMASK = (1 << DBITS) - 1
RPT = 624
OTAIL = N - NS * RPT

BLK = 2000
NBLK = N // BLK


def _rel9_body(w_ref, wc_ref, lw_ref, out_ref):
    for r in range(R):
        acc = wc_ref[r, 0] * w_ref[0]
        for b in range(1, NBASES):
            acc = acc + wc_ref[r, b] * w_ref[b]
        out_ref[r] = acc
    out_ref[R] = lw_ref[...]


def _rel9(weight, w_comp, loop_weight):
    return pl.pallas_call(
        _rel9_body,
        out_shape=jax.ShapeDtypeStruct((R + 1, F, F), jnp.float32),
        in_specs=[
            pl.BlockSpec(memory_space=pltpu.MemorySpace.VMEM),
            pl.BlockSpec(memory_space=pltpu.MemorySpace.SMEM),
            pl.BlockSpec(memory_space=pltpu.MemorySpace.VMEM),
        ],
        out_specs=pl.BlockSpec(memory_space=pltpu.MemorySpace.VMEM),
    )(weight, w_comp, loop_weight)


def _hall_body(x_ref, rel_ref, out_ref):
    out_ref[0] = jnp.dot(x_ref[...], rel_ref[0],
                         preferred_element_type=jnp.float32)


def _hall(x, rel9):
    return pl.pallas_call(
        _hall_body,
        grid=(NBLK, R + 1),
        in_specs=[
            pl.BlockSpec((BLK, F), lambda nb, r: (nb, 0)),
            pl.BlockSpec((1, F, F), lambda nb, r: (r, 0, 0)),
        ],
        out_specs=pl.BlockSpec((1, BLK, F), lambda nb, r: (r, nb, 0)),
        out_shape=jax.ShapeDtypeStruct((R + 1, N, F), jnp.float32),
    )(x, rel9)


def _sc_agg_body(hall_hbm, comb_hbm, zeros_hbm, out_hbm,
                 comb, gidx2, didx2, buf, acc, gsem):
    c = lax.axis_index("c")
    s = lax.axis_index("s")
    tile = c * NS + s

    pltpu.sync_copy(zeros_hbm.at[pl.ds(s * RPT, RPT)],
                    acc.at[pl.ds(s * RPT, RPT)])

    @pl.when(s == NS - 1)
    def _zero_tail():
        pltpu.sync_copy(zeros_hbm.at[pl.ds(NS * RPT, OTAIL)],
                        acc.at[pl.ds(NS * RPT, OTAIL)])

    pltpu.sync_copy(comb_hbm.at[tile], comb)

    def unpack_row(j, slot):
        def uv(k, _):
            sl = pl.ds(k * 16, 16)
            w = comb[j, sl]
            didx2[slot, sl] = lax.bitwise_and(w, DMASK)
            gidx2[slot, sl] = lax.shift_right_logical(w, DBITS)
            return 0
        lax.fori_loop(0, CH // 16, uv, 0)

    plsc.subcore_barrier()

    unpack_row(0, 0)
    pltpu.async_copy(hall_hbm.at[gidx2.at[0]], buf.at[0], gsem)

    def pair(i, _):
        for b in range(2):
            j = 2 * i + b
            o = 1 - b

            @pl.when(j + 1 < NCH)
            def _prefetch():
                unpack_row(j + 1, o)
                pltpu.async_copy(hall_hbm.at[gidx2.at[o]], buf.at[o], gsem)

            pltpu.make_async_copy(hall_hbm.at[gidx2.at[b]], buf.at[b],
                                  gsem).wait()
            pltpu.sync_copy(buf.at[b], acc.at[didx2.at[b]], add=True)
        return 0

    lax.fori_loop(0, NCH // 2, pair, 0)

    last_b = (NCH - 1) % 2
    pltpu.make_async_copy(hall_hbm.at[gidx2.at[last_b]], buf.at[last_b],
                          gsem).wait()
    pltpu.sync_copy(buf.at[last_b], acc.at[didx2.at[last_b]], add=True)

    plsc.subcore_barrier()
    pltpu.sync_copy(acc.at[pl.ds(s * RPT, RPT)],
                    out_hbm.at[c, pl.ds(s * RPT, RPT)])

    @pl.when(s == NS - 1)
    def _write_tail():
        pltpu.sync_copy(acc.at[pl.ds(NS * RPT, OTAIL)],
                        out_hbm.at[c, pl.ds(NS * RPT, OTAIL)])


@functools.cache
def _sc_agg():
    return pl.kernel(
        _sc_agg_body,
        out_type=jax.ShapeDtypeStruct((NC, N, F), jnp.float32),
        mesh=plsc.VectorSubcoreMesh(core_axis_name="c", subcore_axis_name="s",
                                    num_cores=NC, num_subcores=NS),
        scratch_types=[
            pltpu.VMEM((NCH, CH), jnp.int32),
            pltpu.VMEM((2, CH), jnp.int32),
            pltpu.VMEM((2, CH), jnp.int32),
            pltpu.VMEM((2, CH, F), jnp.float32),
            pltpu.VMEM_SHARED((N, F), jnp.float32),
            pltpu.SemaphoreType.DMA,
        ],
    )


def _mlp_body(p_ref, self_ref, bias_ref, w1_ref, b1_ref, w2_ref, b2_ref,
              out_ref):
    h = p_ref[0] + p_ref[1] + self_ref[0] + bias_ref[...]
    h = jnp.maximum(
        jnp.dot(h, w1_ref[...], preferred_element_type=jnp.float32)
        + b1_ref[...], 0.0)
    out_ref[...] = jnp.maximum(
        jnp.dot(h, w2_ref[...], preferred_element_type=jnp.float32)
        + b2_ref[...], 0.0)


def _mlp(parts, hall, h_bias, W1, b1, W2, b2):
    return pl.pallas_call(
        _mlp_body,
        grid=(NBLK,),
        in_specs=[
            pl.BlockSpec((NC, BLK, F), lambda nb: (0, nb, 0)),
            pl.BlockSpec((1, BLK, F), lambda nb: (R, nb, 0)),
            pl.BlockSpec((1, F), lambda nb: (0, 0)),
            pl.BlockSpec((F, F), lambda nb: (0, 0)),
            pl.BlockSpec((1, F), lambda nb: (0, 0)),
            pl.BlockSpec((F, F), lambda nb: (0, 0)),
            pl.BlockSpec((1, F), lambda nb: (0, 0)),
        ],
        out_specs=pl.BlockSpec((BLK, F), lambda nb: (nb, 0)),
        out_shape=jax.ShapeDtypeStruct((N, F), jnp.float32),
    )(parts, hall, h_bias.reshape(1, F), W1, b1.reshape(1, F), W2,
      b2.reshape(1, F))


def kernel(x, edge_index, etypes, weight, w_comp, loop_weight, h_bias,
           W1, b1, W2, b2):
    src = edge_index[0].astype(jnp.int32)
    dst = edge_index[1].astype(jnp.int32)
    et = etypes.astype(jnp.int32)
    comb3 = (((et * N + src) << DBITS) | dst).reshape(NTILES, NCH, CH)

    rel9 = _rel9(weight, w_comp, loop_weight)
    hall = _hall(x, rel9)
    zeros = jnp.zeros((N, F), jnp.float32)
    parts = _sc_agg()(hall.reshape((R + 1) * N, F), comb3, zeros)
    return _mlp(parts, hall, h_bias, W1, b1, W2, b2)


# speedup vs baseline: 3.1779x; 1.0395x over previous
import functools

import jax
import jax.numpy as jnp
from jax import lax
from jax.experimental import pallas as pl
from jax.experimental.pallas import tpu as pltpu
from jax.experimental.pallas import tpu_sc as plsc

N = 10000
E = 320000
F = 128
R = 8
NBASES = 4

NC = 2
NS = 16
NTILES = NC * NS
CH = 80
NCH = 125
EPT = NCH * CH
DBITS = 14
DMASK = (1 << DBITS) - 1
RPT = 624
OTAIL = N - NS * RPT

BLK = 2000
NBLK = N // BLK


def _hall_body(x_ref, w_ref, wc_ref, out_ref):
    r = pl.program_id(1)
    relw = wc_ref[r, 0] * w_ref[0]
    for b in range(1, NBASES):
        relw = relw + wc_ref[r, b] * w_ref[b]
    out_ref[0] = jnp.dot(x_ref[...], relw,
                         preferred_element_type=jnp.float32)


def _hall(x, weight, w_comp):
    return pl.pallas_call(
        _hall_body,
        grid=(NBLK, R),
        in_specs=[
            pl.BlockSpec((BLK, F), lambda nb, r: (nb, 0)),
            pl.BlockSpec(memory_space=pltpu.MemorySpace.VMEM),
            pl.BlockSpec(memory_space=pltpu.MemorySpace.SMEM),
        ],
        out_specs=pl.BlockSpec((1, BLK, F), lambda nb, r: (r, nb, 0)),
        out_shape=jax.ShapeDtypeStruct((R, N, F), jnp.float32),
    )(x, weight, w_comp)


def _sc_agg_body(hall_hbm, comb_hbm, zeros_hbm, out_hbm,
                 comb, gidx2, didx2, buf, acc, gsem):
    c = lax.axis_index("c")
    s = lax.axis_index("s")
    tile = c * NS + s

    pltpu.sync_copy(zeros_hbm.at[pl.ds(s * RPT, RPT)],
                    acc.at[pl.ds(s * RPT, RPT)])

    @pl.when(s == NS - 1)
    def _zero_tail():
        pltpu.sync_copy(zeros_hbm.at[pl.ds(NS * RPT, OTAIL)],
                        acc.at[pl.ds(NS * RPT, OTAIL)])

    pltpu.sync_copy(comb_hbm.at[tile], comb)

    def unpack_row(j, slot):
        def uv(k, _):
            sl = pl.ds(k * 16, 16)
            w = comb[j, sl]
            didx2[slot, sl] = lax.bitwise_and(w, DMASK)
            gidx2[slot, sl] = lax.shift_right_logical(w, DBITS)
            return 0
        lax.fori_loop(0, CH // 16, uv, 0)

    plsc.subcore_barrier()

    unpack_row(0, 0)
    pltpu.async_copy(hall_hbm.at[gidx2.at[0]], buf.at[0], gsem)

    def pair(i, _):
        for b in range(2):
            j = 2 * i + b
            o = 1 - b

            @pl.when(j + 1 < NCH)
            def _prefetch():
                unpack_row(j + 1, o)
                pltpu.async_copy(hall_hbm.at[gidx2.at[o]], buf.at[o], gsem)

            pltpu.make_async_copy(hall_hbm.at[gidx2.at[b]], buf.at[b],
                                  gsem).wait()
            pltpu.sync_copy(buf.at[b], acc.at[didx2.at[b]], add=True)
        return 0

    lax.fori_loop(0, NCH // 2, pair, 0)

    last_b = (NCH - 1) % 2
    pltpu.make_async_copy(hall_hbm.at[gidx2.at[last_b]], buf.at[last_b],
                          gsem).wait()
    pltpu.sync_copy(buf.at[last_b], acc.at[didx2.at[last_b]], add=True)

    plsc.subcore_barrier()
    pltpu.sync_copy(acc.at[pl.ds(s * RPT, RPT)],
                    out_hbm.at[c, pl.ds(s * RPT, RPT)])

    @pl.when(s == NS - 1)
    def _write_tail():
        pltpu.sync_copy(acc.at[pl.ds(NS * RPT, OTAIL)],
                        out_hbm.at[c, pl.ds(NS * RPT, OTAIL)])


@functools.cache
def _sc_agg():
    return pl.kernel(
        _sc_agg_body,
        out_type=jax.ShapeDtypeStruct((NC, N, F), jnp.float32),
        mesh=plsc.VectorSubcoreMesh(core_axis_name="c", subcore_axis_name="s",
                                    num_cores=NC, num_subcores=NS),
        scratch_types=[
            pltpu.VMEM((NCH, CH), jnp.int32),
            pltpu.VMEM((2, CH), jnp.int32),
            pltpu.VMEM((2, CH), jnp.int32),
            pltpu.VMEM((2, CH, F), jnp.float32),
            pltpu.VMEM_SHARED((N, F), jnp.float32),
            pltpu.SemaphoreType.DMA,
        ],
    )


def _mlp_body(p_ref, x_ref, lw_ref, bias_ref, w1_ref, b1_ref, w2_ref,
              b2_ref, out_ref):
    h = (p_ref[0] + p_ref[1] + bias_ref[...]
         + jnp.dot(x_ref[...], lw_ref[...],
                   preferred_element_type=jnp.float32))
    h = jnp.maximum(
        jnp.dot(h, w1_ref[...], preferred_element_type=jnp.float32)
        + b1_ref[...], 0.0)
    out_ref[...] = jnp.maximum(
        jnp.dot(h, w2_ref[...], preferred_element_type=jnp.float32)
        + b2_ref[...], 0.0)


def _mlp(parts, x, loop_weight, h_bias, W1, b1, W2, b2):
    return pl.pallas_call(
        _mlp_body,
        grid=(NBLK,),
        in_specs=[
            pl.BlockSpec((NC, BLK, F), lambda nb: (0, nb, 0)),
            pl.BlockSpec((BLK, F), lambda nb: (nb, 0)),
            pl.BlockSpec((F, F), lambda nb: (0, 0)),
            pl.BlockSpec((1, F), lambda nb: (0, 0)),
            pl.BlockSpec((F, F), lambda nb: (0, 0)),
            pl.BlockSpec((1, F), lambda nb: (0, 0)),
            pl.BlockSpec((F, F), lambda nb: (0, 0)),
            pl.BlockSpec((1, F), lambda nb: (0, 0)),
        ],
        out_specs=pl.BlockSpec((BLK, F), lambda nb: (nb, 0)),
        out_shape=jax.ShapeDtypeStruct((N, F), jnp.float32),
    )(parts, x, loop_weight, h_bias.reshape(1, F), W1, b1.reshape(1, F),
      W2, b2.reshape(1, F))


def kernel(x, edge_index, etypes, weight, w_comp, loop_weight, h_bias,
           W1, b1, W2, b2):
    src = edge_index[0].astype(jnp.int32)
    dst = edge_index[1].astype(jnp.int32)
    et = etypes.astype(jnp.int32)
    comb3 = (((et * N + src) << DBITS) | dst).reshape(NTILES, NCH, CH)

    hall = _hall(x, weight, w_comp)
    zeros = jnp.zeros((N, F), jnp.float32)
    parts = _sc_agg()(hall.reshape(R * N, F), comb3, zeros)
    return _mlp(parts, x, loop_weight, h_bias, W1, b1, W2, b2)
